# baseline (speedup 1.0000x reference)
import jax
import jax.numpy as jnp
from jax import lax
from jax.experimental import pallas as pl
from jax.experimental.pallas import tpu as pltpu
from jax.experimental.pallas import tpu_sc as plsc

_NC, _NS, _L = 2, 16, 16
_NW = _NC * _NS


def _tc_body(x_ref, rm_ref, idx_ref):
    prod_t = lax.dot_general(
        rm_ref[...], x_ref[...], (((0,), (1,)), ((), ())),
        preferred_element_type=jnp.float32)
    powers = jnp.left_shift(
        jnp.int32(1), lax.broadcasted_iota(jnp.int32, prod_t.shape, 0))
    masked = jnp.where(prod_t < 0.0, powers, 0)
    idx = jnp.sum(masked, axis=0)
    idx_ref[...] = idx.reshape(1, 1, idx.shape[0])


def _gather_body(idx_hbm, tab_hbm, out_hbm, idx_v, widx_v, words_v, out_v, sem):
    wid = lax.axis_index("s") * _NC + lax.axis_index("c")
    bpw = idx_v.shape[0]
    blk = idx_hbm.shape[2]
    per_blk = blk // bpw
    row = wid // per_blk
    off = (wid % per_blk) * bpw
    pltpu.sync_copy(idx_hbm.at[row, 0, pl.ds(off, bpw)], idx_v)
    base = wid * bpw
    nv = bpw // _L
    per_row = widx_v.shape[1] // _L
    for j in range(nv):
        h = idx_v[pl.ds(j * _L, _L)]
        widx_v[j // per_row, pl.ds((j % per_row) * _L, _L)] = (
            lax.shift_right_logical(h, 5))
    copies = []
    for c in range(widx_v.shape[0]):
        cp = pltpu.make_async_copy(
            tab_hbm.at[widx_v.at[c]], words_v.at[c], sem)
        cp.start()
        copies.append(cp)
    for cp in copies:
        cp.wait()
    for j in range(nv):
        w = words_v[j // per_row, pl.ds((j % per_row) * _L, _L)]
        h = idx_v[pl.ds(j * _L, _L)]
        bit = lax.shift_right_logical(w, jnp.bitwise_and(h, 31)) & 1
        out_v[pl.ds(j * _L, _L)] = bit
    pltpu.sync_copy(out_v, out_hbm.at[0, pl.ds(base, bpw)])


def kernel(x, random_matrix, binary_set):
    B, D = x.shape
    nbits = random_matrix.shape[1]
    nbytes = binary_set.shape[0]
    blk = 2048
    ng = B // blk
    idx3 = pl.pallas_call(
        _tc_body,
        grid=(ng,),
        in_specs=[
            pl.BlockSpec((blk, D), lambda i: (i, 0)),
            pl.BlockSpec((D, nbits), lambda i: (0, 0)),
        ],
        out_specs=pl.BlockSpec((1, 1, blk), lambda i: (i, 0, 0)),
        out_shape=jax.ShapeDtypeStruct((ng, 1, blk), jnp.int32),
    )(x, random_matrix)
    table32 = lax.bitcast_convert_type(
        binary_set.reshape(nbytes // 4, 4), jnp.int32)

    bpw = B // _NW
    mesh = plsc.VectorSubcoreMesh(core_axis_name="c", subcore_axis_name="s")
    gather = pl.kernel(
        _gather_body,
        out_type=jax.ShapeDtypeStruct((1, B), jnp.bool_),
        mesh=mesh,
        scratch_types=[
            pltpu.VMEM((bpw,), jnp.int32),
            pltpu.VMEM((bpw // 128, 128), jnp.int32),
            pltpu.VMEM((bpw // 128, 128), jnp.int32),
            pltpu.VMEM((bpw,), jnp.int32),
            pltpu.SemaphoreType.DMA,
        ],
    )
    return gather(idx3, table32).reshape(B)


# speedup vs baseline: 13.7659x; 13.7659x over previous
import jax
import jax.numpy as jnp
from jax import lax
from jax.experimental import pallas as pl
from jax.experimental.pallas import tpu as pltpu
from jax.experimental.pallas import tpu_sc as plsc

_NC, _NS, _L = 2, 16, 16
_NW = _NC * _NS


def _tc_body(x_ref, rm_ref, b_ref, idx_ref, w_ref):
    prod_t = lax.dot_general(
        rm_ref[...], x_ref[...], (((0,), (1,)), ((), ())),
        preferred_element_type=jnp.float32)
    powers = jnp.left_shift(
        jnp.int32(1), lax.broadcasted_iota(jnp.int32, prod_t.shape, 0))
    masked = jnp.where(prod_t < 0.0, powers, 0)
    idx = jnp.sum(masked, axis=0)
    idx_ref[...] = idx.reshape(1, 1, idx.shape[0])
    b2d = b_ref[...].reshape(b_ref.shape[0] // 128, 128)
    w_ref[...] = pltpu.bitcast(b2d, jnp.int32)


def _gather_body(idx_hbm, tab_hbm, out_hbm, idx_v, widx_v, words_v, out_v, sem):
    wid = lax.axis_index("s") * _NC + lax.axis_index("c")
    bpw = idx_v.shape[0]
    blk = idx_hbm.shape[2]
    per_blk = blk // bpw
    row = wid // per_blk
    off = (wid % per_blk) * bpw
    pltpu.sync_copy(idx_hbm.at[row, 0, pl.ds(off, bpw)], idx_v)
    base = wid * bpw
    nv = bpw // _L
    per_row = widx_v.shape[1] // _L
    for j in range(nv):
        h = idx_v[pl.ds(j * _L, _L)]
        p = jnp.bitwise_or(
            jnp.left_shift(lax.shift_right_logical(h, 12), 7),
            jnp.bitwise_and(lax.shift_right_logical(h, 3), 127))
        widx_v[j // per_row, pl.ds((j % per_row) * _L, _L)] = p
    copies = []
    for c in range(widx_v.shape[0]):
        cp = pltpu.make_async_copy(
            tab_hbm.at[widx_v.at[c]], words_v.at[c], sem)
        cp.start()
        copies.append(cp)
    for cp in copies:
        cp.wait()
    for j in range(nv):
        w = words_v[j // per_row, pl.ds((j % per_row) * _L, _L)]
        h = idx_v[pl.ds(j * _L, _L)]
        shift = jnp.bitwise_or(
            jnp.left_shift(jnp.bitwise_and(lax.shift_right_logical(h, 10), 3), 3),
            jnp.bitwise_and(h, 7))
        bit = lax.shift_right_logical(w, shift) & 1
        out_v[pl.ds(j * _L, _L)] = bit
    pltpu.sync_copy(out_v, out_hbm.at[0, pl.ds(base, bpw)])


def kernel(x, random_matrix, binary_set):
    B, D = x.shape
    nbits = random_matrix.shape[1]
    nbytes = binary_set.shape[0]
    blk = 4096
    ng = B // blk
    rblk = nbytes // ng
    idx3, table2d = pl.pallas_call(
        _tc_body,
        grid=(ng,),
        in_specs=[
            pl.BlockSpec((blk, D), lambda i: (i, 0)),
            pl.BlockSpec((D, nbits), lambda i: (0, 0)),
            pl.BlockSpec((rblk,), lambda i: (i,)),
        ],
        out_specs=[
            pl.BlockSpec((1, 1, blk), lambda i: (i, 0, 0)),
            pl.BlockSpec((rblk // 512, 128), lambda i: (i, 0)),
        ],
        out_shape=[
            jax.ShapeDtypeStruct((ng, 1, blk), jnp.int32),
            jax.ShapeDtypeStruct((nbytes // 512, 128), jnp.int32),
        ],
    )(x, random_matrix, binary_set)
    table32 = table2d.reshape(nbytes // 4)

    bpw = B // _NW
    mesh = plsc.VectorSubcoreMesh(core_axis_name="c", subcore_axis_name="s")
    gather = pl.kernel(
        _gather_body,
        out_type=jax.ShapeDtypeStruct((1, B), jnp.bool_),
        mesh=mesh,
        scratch_types=[
            pltpu.VMEM((bpw,), jnp.int32),
            pltpu.VMEM((bpw // 128, 128), jnp.int32),
            pltpu.VMEM((bpw // 128, 128), jnp.int32),
            pltpu.VMEM((bpw,), jnp.int32),
            pltpu.SemaphoreType.DMA,
        ],
    )
    return gather(idx3, table32).reshape(B)


# speedup vs baseline: 13.9733x; 1.0151x over previous
import jax
import jax.numpy as jnp
from jax import lax
from jax.experimental import pallas as pl
from jax.experimental.pallas import tpu as pltpu
from jax.experimental.pallas import tpu_sc as plsc

_NC, _NS, _L = 2, 16, 16
_NW = _NC * _NS


def _tc_body(x_ref, rm_ref, b_ref, idx_ref, w_ref):
    prod_t = lax.dot_general(
        rm_ref[...], x_ref[...], (((0,), (1,)), ((), ())),
        preferred_element_type=jnp.float32)
    powers = jnp.left_shift(
        jnp.int32(1), lax.broadcasted_iota(jnp.int32, prod_t.shape, 0))
    masked = jnp.where(prod_t < 0.0, powers, 0)
    idx = jnp.sum(masked, axis=0)
    idx_ref[...] = idx.reshape(1, 1, idx.shape[0])
    b2d = b_ref[...].reshape(b_ref.shape[0] // 128, 128)
    w_ref[...] = pltpu.bitcast(b2d, jnp.int32)


def _gather_body(idx_hbm, tab_hbm, out_hbm, idx_v, widx_v, words_v, out_v, sem):
    wid = lax.axis_index("s") * _NC + lax.axis_index("c")
    bpw = idx_v.shape[0]
    blk = idx_hbm.shape[2]
    per_blk = blk // bpw
    row = wid // per_blk
    off = (wid % per_blk) * bpw
    pltpu.sync_copy(idx_hbm.at[row, 0, pl.ds(off, bpw)], idx_v)
    base = wid * bpw
    nv = bpw // _L
    per_row = widx_v.shape[1] // _L
    for j in range(nv):
        h = idx_v[pl.ds(j * _L, _L)]
        p = jnp.bitwise_or(
            jnp.left_shift(lax.shift_right_logical(h, 12), 7),
            jnp.bitwise_and(lax.shift_right_logical(h, 3), 127))
        widx_v[j // per_row, pl.ds((j % per_row) * _L, _L)] = p
    copies = []
    for c in range(widx_v.shape[0]):
        cp = pltpu.make_async_copy(
            tab_hbm.at[widx_v.at[c]], words_v.at[c], sem)
        cp.start()
        copies.append(cp)
    for cp in copies:
        cp.wait()
    for j in range(nv):
        w = words_v[j // per_row, pl.ds((j % per_row) * _L, _L)]
        h = idx_v[pl.ds(j * _L, _L)]
        shift = jnp.bitwise_or(
            jnp.left_shift(jnp.bitwise_and(lax.shift_right_logical(h, 10), 3), 3),
            jnp.bitwise_and(h, 7))
        bit = lax.shift_right_logical(w, shift) & 1
        out_v[pl.ds(j * _L, _L)] = bit
    pltpu.sync_copy(out_v, out_hbm.at[0, pl.ds(base, bpw)])


def kernel(x, random_matrix, binary_set):
    B, D = x.shape
    nbits = random_matrix.shape[1]
    nbytes = binary_set.shape[0]
    blk = 8192
    ng = B // blk
    rblk = nbytes // ng
    idx3, table2d = pl.pallas_call(
        _tc_body,
        grid=(ng,),
        in_specs=[
            pl.BlockSpec((blk, D), lambda i: (i, 0)),
            pl.BlockSpec((D, nbits), lambda i: (0, 0)),
            pl.BlockSpec((rblk,), lambda i: (i,)),
        ],
        out_specs=[
            pl.BlockSpec((1, 1, blk), lambda i: (i, 0, 0)),
            pl.BlockSpec((rblk // 512, 128), lambda i: (i, 0)),
        ],
        out_shape=[
            jax.ShapeDtypeStruct((ng, 1, blk), jnp.int32),
            jax.ShapeDtypeStruct((nbytes // 512, 128), jnp.int32),
        ],
    )(x, random_matrix, binary_set)
    table32 = table2d.reshape(nbytes // 4)

    bpw = B // _NW
    mesh = plsc.VectorSubcoreMesh(core_axis_name="c", subcore_axis_name="s")
    gather = pl.kernel(
        _gather_body,
        out_type=jax.ShapeDtypeStruct((1, B), jnp.bool_),
        mesh=mesh,
        scratch_types=[
            pltpu.VMEM((bpw,), jnp.int32),
            pltpu.VMEM((bpw // 128, 128), jnp.int32),
            pltpu.VMEM((bpw // 128, 128), jnp.int32),
            pltpu.VMEM((bpw,), jnp.int32),
            pltpu.SemaphoreType.DMA,
        ],
    )
    return gather(idx3, table32).reshape(B)
